# baseline (speedup 1.0000x reference)
import functools

import jax
import jax.numpy as jnp
from jax import lax
from jax.experimental import pallas as pl
from jax.experimental.pallas import tpu as pltpu
from jax.experimental.pallas import tpu_sc as plsc

BATCH = 4096
FIELDS = 26
EPS = 1e-5
VOCAB_ = 1000000

_NC = 2
_NS = 16
_NW = _NC * _NS
_N_PER_W = BATCH * FIELDS // _NW
_CHUNK = 128
_NCHUNK = _N_PER_W // _CHUNK


def _gather_body(x_hbm, table_hbm, out_hbm, xv, embv, sem):
    wid = lax.axis_index("s") * _NC + lax.axis_index("c")
    base = wid * _N_PER_W
    pltpu.sync_copy(x_hbm.at[pl.ds(base, _N_PER_W)], xv)

    def fire(j, carry):
        pltpu.async_copy(
            table_hbm.at[xv.at[pl.ds(j * _CHUNK, _CHUNK)]],
            embv.at[pl.ds(j * _CHUNK, _CHUNK)],
            sem,
        )
        return carry

    lax.fori_loop(0, _NCHUNK, fire, 0)

    def drain(j, carry):
        pltpu.make_async_copy(
            table_hbm.at[xv.at[pl.ds(j * _CHUNK, _CHUNK)]],
            embv.at[pl.ds(j * _CHUNK, _CHUNK)],
            sem,
        ).wait()
        return carry

    lax.fori_loop(0, _NCHUNK, drain, 0)
    pltpu.sync_copy(embv, out_hbm.at[pl.ds(base, _N_PER_W)])


_gather = functools.partial(
    pl.kernel,
    mesh=plsc.VectorSubcoreMesh(core_axis_name="c", subcore_axis_name="s"),
    out_type=jax.ShapeDtypeStruct((BATCH * FIELDS,), jnp.float32),
    scratch_types=[
        pltpu.VMEM((_N_PER_W,), jnp.int32),
        pltpu.VMEM((_N_PER_W,), jnp.float32),
        pltpu.SemaphoreType.DMA,
    ],
)(_gather_body)


def _flatten_body(src_ref, dst_ref, sem):
    copy = pltpu.make_async_copy(src_ref.at[0], dst_ref, sem)
    copy.start()
    copy.wait()


def _flatten(table2d):
    return pl.pallas_call(
        _flatten_body,
        in_specs=[pl.BlockSpec(memory_space=pl.ANY)],
        out_specs=pl.BlockSpec(memory_space=pl.ANY),
        out_shape=jax.ShapeDtypeStruct((VOCAB_,), jnp.float32),
        scratch_shapes=[pltpu.SemaphoreType.DMA],
    )(table2d)


def _finish_body(emb_ref, gamma_ref, beta_ref, out_ref):
    emb = emb_ref[...]
    gamma = gamma_ref[...]
    beta = beta_ref[...]
    mean = jnp.mean(emb, axis=0, keepdims=True)
    var = jnp.mean((emb - mean) * (emb - mean), axis=0, keepdims=True)
    w = gamma * lax.rsqrt(var + EPS)
    c = jnp.sum(beta - w * mean)
    z = jnp.sum(emb * w, axis=1, keepdims=True) + c
    out_ref[...] = 1.0 / (1.0 + jnp.exp(-z))


def _finish(emb, gamma, beta):
    return pl.pallas_call(
        _finish_body,
        out_shape=jax.ShapeDtypeStruct((BATCH, 1), jnp.float32),
    )(emb, gamma, beta)


@jax.jit
def kernel(x, table, gamma, beta):
    x_flat = x.reshape(-1).astype(jnp.int32)
    table_flat = _flatten(table.reshape(1, VOCAB_))
    emb = _gather(x_flat, table_flat).reshape(BATCH, FIELDS)
    return _finish(emb, gamma.reshape(1, FIELDS), beta.reshape(1, FIELDS))


# speedup vs baseline: 4.8468x; 4.8468x over previous
import functools

import jax
import jax.numpy as jnp
from jax import lax
from jax.experimental import pallas as pl
from jax.experimental.pallas import tpu as pltpu
from jax.experimental.pallas import tpu_sc as plsc

BATCH = 4096
FIELDS = 26
VOCAB = 1000000
EPS = 1e-5

_NC = 2
_NS = 16
_NW = _NC * _NS
_ROWS_W = BATCH // _NW
_N_PER_W = _ROWS_W * FIELDS
_LANES = 16


def _flatten_body(src_ref, dst_ref):
    dst_ref[...] = src_ref[0, :]


_FLAT_BLK = 131072


def _flatten(table2d):
    return pl.pallas_call(
        _flatten_body,
        grid=(8,),
        in_specs=[pl.BlockSpec((1, _FLAT_BLK), lambda i: (0, i))],
        out_specs=pl.BlockSpec((_FLAT_BLK,), lambda i: (i,)),
        out_shape=jax.ShapeDtypeStruct((VOCAB,), jnp.float32),
    )(pltpu.with_memory_space_constraint(table2d, pltpu.MemorySpace.HBM))


def _gather_body(xt_hbm, table_hbm, out_hbm, xtv, embv, sem):
    wid = lax.axis_index("s") * _NC + lax.axis_index("c")
    pltpu.sync_copy(xt_hbm.at[:, wid], xtv)

    def fire(f, carry):
        pltpu.async_copy(table_hbm.at[xtv.at[f]], embv.at[f], sem)
        return carry

    lax.fori_loop(0, FIELDS, fire, 0)

    def drain(f, carry):
        pltpu.make_async_copy(table_hbm.at[xtv.at[f]], embv.at[f], sem).wait()
        return carry

    lax.fori_loop(0, FIELDS, drain, 0)
    pltpu.sync_copy(embv, out_hbm.at[:, wid])


_gather = functools.partial(
    pl.kernel,
    mesh=plsc.VectorSubcoreMesh(core_axis_name="c", subcore_axis_name="s"),
    out_type=jax.ShapeDtypeStruct((FIELDS, _NW, _ROWS_W), jnp.float32),
    scratch_types=[
        pltpu.VMEM((FIELDS, _ROWS_W), jnp.int32),
        pltpu.VMEM((FIELDS, _ROWS_W), jnp.float32),
        pltpu.SemaphoreType.DMA,
    ],
)(_gather_body)


def _finish_body(emb_ref, gamma_ref, beta_ref, out_ref):
    emb = emb_ref[...].reshape(FIELDS, _NW, _ROWS_W)
    gamma = gamma_ref[...].reshape(FIELDS)
    beta = beta_ref[...].reshape(FIELDS)
    sums = jnp.sum(emb, axis=(1, 2))
    sqs = jnp.sum(emb * emb, axis=(1, 2))
    mean = sums * (1.0 / BATCH)
    var = sqs * (1.0 / BATCH) - mean * mean
    w = gamma * lax.rsqrt(var + EPS)
    c = jnp.sum(beta - w * mean)
    z = jnp.sum(emb * w[:, None, None], axis=0) + c
    out_ref[...] = 1.0 / (1.0 + jnp.exp(-z))


def _finish(emb832, gamma, beta):
    return pl.pallas_call(
        _finish_body,
        out_shape=jax.ShapeDtypeStruct((_NW, _ROWS_W), jnp.float32),
    )(emb832, gamma, beta)


@jax.jit
def kernel(x, table, gamma, beta):
    xt3 = x.T.astype(jnp.int32).reshape(FIELDS, _NW, _ROWS_W)
    table_flat = _flatten(table.reshape(1, VOCAB))
    emb = _gather(xt3, table_flat)
    emb832 = emb.reshape(FIELDS * _NW, _ROWS_W)
    out = _finish(emb832, gamma.reshape(1, FIELDS), beta.reshape(1, FIELDS))
    return out.reshape(BATCH, 1)


# speedup vs baseline: 4.8898x; 1.0089x over previous
import functools

import jax
import jax.numpy as jnp
from jax import lax
from jax.experimental import pallas as pl
from jax.experimental.pallas import tpu as pltpu
from jax.experimental.pallas import tpu_sc as plsc

BATCH = 4096
FIELDS = 26
VOCAB = 1000000
EPS = 1e-5

_NC = 2
_NS = 16
_NW = _NC * _NS
_ROWS_W = BATCH // _NW
_N_PER_W = _ROWS_W * FIELDS
_LANES = 16


def _flatten_body(src_ref, dst_ref):
    dst_ref[...] = src_ref[0, :]


_FLAT_BLK = 131072


def _flatten(table2d):
    return pl.pallas_call(
        _flatten_body,
        grid=(8,),
        in_specs=[pl.BlockSpec((1, _FLAT_BLK), lambda i: (0, i))],
        out_specs=pl.BlockSpec((_FLAT_BLK,), lambda i: (i,)),
        out_shape=jax.ShapeDtypeStruct((VOCAB,), jnp.float32),
    )(pltpu.with_memory_space_constraint(table2d, pltpu.MemorySpace.HBM))


def _gather_body(xt_hbm, table_hbm, out_hbm, xtv, embv, sem):
    wid = lax.axis_index("s") * _NC + lax.axis_index("c")
    base = wid * _N_PER_W
    pltpu.sync_copy(xt_hbm.at[pl.ds(base, _N_PER_W)], xtv)
    pltpu.async_copy(table_hbm.at[xtv], embv, sem).wait()
    pltpu.sync_copy(embv, out_hbm.at[pl.ds(base, _N_PER_W)])


_gather = functools.partial(
    pl.kernel,
    mesh=plsc.VectorSubcoreMesh(core_axis_name="c", subcore_axis_name="s"),
    out_type=jax.ShapeDtypeStruct((BATCH * FIELDS,), jnp.float32),
    scratch_types=[
        pltpu.VMEM((_N_PER_W,), jnp.int32),
        pltpu.VMEM((_N_PER_W,), jnp.float32),
        pltpu.SemaphoreType.DMA,
    ],
)(_gather_body)


def _finish_body(emb_ref, gamma_ref, beta_ref, out_ref):
    emb = emb_ref[...].reshape(FIELDS, _NW, _ROWS_W)
    gamma = gamma_ref[...].reshape(FIELDS)
    beta = beta_ref[...].reshape(FIELDS)
    sums = jnp.sum(emb, axis=(1, 2))
    sqs = jnp.sum(emb * emb, axis=(1, 2))
    mean = sums * (1.0 / BATCH)
    var = sqs * (1.0 / BATCH) - mean * mean
    w = gamma * lax.rsqrt(var + EPS)
    c = jnp.sum(beta - w * mean)
    z = jnp.sum(emb * w[:, None, None], axis=0) + c
    out_ref[...] = 1.0 / (1.0 + jnp.exp(-z))


def _finish(emb832, gamma, beta):
    return pl.pallas_call(
        _finish_body,
        out_shape=jax.ShapeDtypeStruct((_NW, _ROWS_W), jnp.float32),
    )(emb832, gamma, beta)


@jax.jit
def kernel(x, table, gamma, beta):
    xt_flat = x.T.astype(jnp.int32).reshape(BATCH * FIELDS)
    table_flat = _flatten(table.reshape(1, VOCAB))
    emb = _gather(xt_flat, table_flat)
    emb832 = emb.reshape(FIELDS * _NW, _ROWS_W)
    out = _finish(emb832, gamma.reshape(1, FIELDS), beta.reshape(1, FIELDS))
    return out.reshape(BATCH, 1)


# speedup vs baseline: 5.0414x; 1.0310x over previous
import functools

import jax
import jax.numpy as jnp
from jax import lax
from jax.experimental import pallas as pl
from jax.experimental.pallas import tpu as pltpu
from jax.experimental.pallas import tpu_sc as plsc

BATCH = 4096
FIELDS = 26
VOCAB = 1000000
EPS = 1e-5

_NC = 2
_NS = 16
_NW = _NC * _NS
_ROWS_W = BATCH // _NW
_N_PER_W = _ROWS_W * FIELDS
_LANES = 16


def _flatten_body(src_ref, dst_ref):
    dst_ref[...] = src_ref[0, :]


_FLAT_BLK = 65536


def _flatten(table2d):
    return pl.pallas_call(
        _flatten_body,
        grid=(8,),
        in_specs=[pl.BlockSpec((1, _FLAT_BLK), lambda i: (0, i))],
        out_specs=pl.BlockSpec((_FLAT_BLK,), lambda i: (i,)),
        out_shape=jax.ShapeDtypeStruct((VOCAB,), jnp.float32),
    )(pltpu.with_memory_space_constraint(table2d, pltpu.MemorySpace.HBM))


def _gather_body(xt_hbm, table_hbm, out_hbm, xtv, embv, sem):
    wid = lax.axis_index("s") * _NC + lax.axis_index("c")
    base = wid * _N_PER_W
    pltpu.sync_copy(xt_hbm.at[pl.ds(base, _N_PER_W)], xtv)
    pltpu.async_copy(table_hbm.at[xtv], embv, sem).wait()
    pltpu.sync_copy(embv, out_hbm.at[pl.ds(base, _N_PER_W)])


_gather = functools.partial(
    pl.kernel,
    mesh=plsc.VectorSubcoreMesh(core_axis_name="c", subcore_axis_name="s"),
    out_type=jax.ShapeDtypeStruct((BATCH * FIELDS,), jnp.float32),
    scratch_types=[
        pltpu.VMEM((_N_PER_W,), jnp.int32),
        pltpu.VMEM((_N_PER_W,), jnp.float32),
        pltpu.SemaphoreType.DMA,
    ],
)(_gather_body)


def _finish_body(emb_ref, gamma_ref, beta_ref, out_ref):
    emb = emb_ref[...].reshape(FIELDS, _NW, _ROWS_W)
    gamma = gamma_ref[...].reshape(FIELDS)
    beta = beta_ref[...].reshape(FIELDS)
    sums = jnp.sum(emb, axis=(1, 2))
    sqs = jnp.sum(emb * emb, axis=(1, 2))
    mean = sums * (1.0 / BATCH)
    var = sqs * (1.0 / BATCH) - mean * mean
    w = gamma * lax.rsqrt(var + EPS)
    c = jnp.sum(beta - w * mean)
    z = jnp.sum(emb * w[:, None, None], axis=0) + c
    out_ref[...] = 1.0 / (1.0 + jnp.exp(-z))


def _finish(emb832, gamma, beta):
    return pl.pallas_call(
        _finish_body,
        out_shape=jax.ShapeDtypeStruct((_NW, _ROWS_W), jnp.float32),
    )(emb832, gamma, beta)


@jax.jit
def kernel(x, table, gamma, beta):
    xt_flat = x.T.astype(jnp.int32).reshape(BATCH * FIELDS)
    table_flat = _flatten(table.reshape(1, VOCAB))
    emb = _gather(xt_flat, table_flat)
    emb832 = emb.reshape(FIELDS * _NW, _ROWS_W)
    out = _finish(emb832, gamma.reshape(1, FIELDS), beta.reshape(1, FIELDS))
    return out.reshape(BATCH, 1)


# speedup vs baseline: 5.2146x; 1.0344x over previous
import functools

import jax
import jax.numpy as jnp
from jax import lax
from jax.experimental import pallas as pl
from jax.experimental.pallas import tpu as pltpu
from jax.experimental.pallas import tpu_sc as plsc

BATCH = 4096
FIELDS = 26
VOCAB = 1000000
EPS = 1e-5

_NC = 2
_NS = 16
_NW = _NC * _NS
_ROWS_W = BATCH // _NW
_N_PER_W = _ROWS_W * FIELDS
_LANES = 16


def _flatten_body(src_ref, dst_ref):
    dst_ref[...] = src_ref[0, :]


_FLAT_BLK = 262144


def _flatten(table2d):
    return pl.pallas_call(
        _flatten_body,
        grid=(4,),
        in_specs=[pl.BlockSpec((1, _FLAT_BLK), lambda i: (0, i))],
        out_specs=pl.BlockSpec((_FLAT_BLK,), lambda i: (i,)),
        out_shape=jax.ShapeDtypeStruct((VOCAB,), jnp.float32),
    )(pltpu.with_memory_space_constraint(table2d, pltpu.MemorySpace.HBM))


def _gather_body(xt_hbm, table_hbm, out_hbm, xtv, embv, sem):
    wid = lax.axis_index("s") * _NC + lax.axis_index("c")
    base = wid * _N_PER_W
    pltpu.sync_copy(xt_hbm.at[pl.ds(base, _N_PER_W)], xtv)
    pltpu.async_copy(table_hbm.at[xtv], embv, sem).wait()
    pltpu.sync_copy(embv, out_hbm.at[pl.ds(base, _N_PER_W)])


_gather = functools.partial(
    pl.kernel,
    mesh=plsc.VectorSubcoreMesh(core_axis_name="c", subcore_axis_name="s"),
    out_type=jax.ShapeDtypeStruct((BATCH * FIELDS,), jnp.float32),
    scratch_types=[
        pltpu.VMEM((_N_PER_W,), jnp.int32),
        pltpu.VMEM((_N_PER_W,), jnp.float32),
        pltpu.SemaphoreType.DMA,
    ],
)(_gather_body)


def _finish_body(emb_ref, gamma_ref, beta_ref, out_ref):
    emb = emb_ref[...].reshape(FIELDS, _NW, _ROWS_W)
    gamma = gamma_ref[...].reshape(FIELDS)
    beta = beta_ref[...].reshape(FIELDS)
    sums = jnp.sum(emb, axis=(1, 2))
    sqs = jnp.sum(emb * emb, axis=(1, 2))
    mean = sums * (1.0 / BATCH)
    var = sqs * (1.0 / BATCH) - mean * mean
    w = gamma * lax.rsqrt(var + EPS)
    c = jnp.sum(beta - w * mean)
    z = jnp.sum(emb * w[:, None, None], axis=0) + c
    out_ref[...] = 1.0 / (1.0 + jnp.exp(-z))


def _finish(emb832, gamma, beta):
    return pl.pallas_call(
        _finish_body,
        out_shape=jax.ShapeDtypeStruct((_NW, _ROWS_W), jnp.float32),
    )(emb832, gamma, beta)


@jax.jit
def kernel(x, table, gamma, beta):
    xt_flat = x.T.astype(jnp.int32).reshape(BATCH * FIELDS)
    table_flat = _flatten(table.reshape(1, VOCAB))
    emb = _gather(xt_flat, table_flat)
    emb832 = emb.reshape(FIELDS * _NW, _ROWS_W)
    out = _finish(emb832, gamma.reshape(1, FIELDS), beta.reshape(1, FIELDS))
    return out.reshape(BATCH, 1)


# speedup vs baseline: 5.4481x; 1.0448x over previous
import functools

import jax
import jax.numpy as jnp
from jax import lax
from jax.experimental import pallas as pl
from jax.experimental.pallas import tpu as pltpu
from jax.experimental.pallas import tpu_sc as plsc

BATCH = 4096
FIELDS = 26
VOCAB = 1000000
EPS = 1e-5

_NC = 2
_NS = 16
_NW = _NC * _NS
_ROWS_W = BATCH // _NW
_N_PER_W = _ROWS_W * FIELDS
_LANES = 16


def _flatten_body(src_ref, dst_ref):
    dst_ref[...] = src_ref[0, :]


_FLAT_BLK = 524288


def _flatten(table2d):
    return pl.pallas_call(
        _flatten_body,
        grid=(2,),
        in_specs=[pl.BlockSpec((1, _FLAT_BLK), lambda i: (0, i))],
        out_specs=pl.BlockSpec((_FLAT_BLK,), lambda i: (i,)),
        out_shape=jax.ShapeDtypeStruct((VOCAB,), jnp.float32),
    )(pltpu.with_memory_space_constraint(table2d, pltpu.MemorySpace.HBM))


def _gather_body(xt_hbm, table_hbm, out_hbm, xtv, embv, sem):
    wid = lax.axis_index("s") * _NC + lax.axis_index("c")
    base = wid * _N_PER_W
    pltpu.sync_copy(xt_hbm.at[pl.ds(base, _N_PER_W)], xtv)
    pltpu.async_copy(table_hbm.at[xtv], embv, sem).wait()
    pltpu.sync_copy(embv, out_hbm.at[pl.ds(base, _N_PER_W)])


_gather = functools.partial(
    pl.kernel,
    mesh=plsc.VectorSubcoreMesh(core_axis_name="c", subcore_axis_name="s"),
    out_type=jax.ShapeDtypeStruct((BATCH * FIELDS,), jnp.float32),
    scratch_types=[
        pltpu.VMEM((_N_PER_W,), jnp.int32),
        pltpu.VMEM((_N_PER_W,), jnp.float32),
        pltpu.SemaphoreType.DMA,
    ],
)(_gather_body)


def _finish_body(emb_ref, gamma_ref, beta_ref, out_ref):
    emb = emb_ref[...].reshape(FIELDS, _NW, _ROWS_W)
    gamma = gamma_ref[...].reshape(FIELDS)
    beta = beta_ref[...].reshape(FIELDS)
    sums = jnp.sum(emb, axis=(1, 2))
    sqs = jnp.sum(emb * emb, axis=(1, 2))
    mean = sums * (1.0 / BATCH)
    var = sqs * (1.0 / BATCH) - mean * mean
    w = gamma * lax.rsqrt(var + EPS)
    c = jnp.sum(beta - w * mean)
    z = jnp.sum(emb * w[:, None, None], axis=0) + c
    out_ref[...] = 1.0 / (1.0 + jnp.exp(-z))


def _finish(emb832, gamma, beta):
    return pl.pallas_call(
        _finish_body,
        out_shape=jax.ShapeDtypeStruct((_NW, _ROWS_W), jnp.float32),
    )(emb832, gamma, beta)


@jax.jit
def kernel(x, table, gamma, beta):
    xt_flat = x.T.astype(jnp.int32).reshape(BATCH * FIELDS)
    table_flat = _flatten(table.reshape(1, VOCAB))
    emb = _gather(xt_flat, table_flat)
    emb832 = emb.reshape(FIELDS * _NW, _ROWS_W)
    out = _finish(emb832, gamma.reshape(1, FIELDS), beta.reshape(1, FIELDS))
    return out.reshape(BATCH, 1)
